# Initial kernel scaffold; baseline (speedup 1.0000x reference)
import jax
import jax.numpy as jnp
from jax.experimental import pallas as pl


def kernel(hidden_states, group_gate_w, mini_gates):
    raise NotImplementedError("write your pallas kernel here")


# speedup vs baseline: 5.8157x; 5.8157x over previous
import jax
import jax.numpy as jnp
from jax.experimental import pallas as pl
from jax.experimental.pallas import tpu as pltpu

T = 8192
D = 1024
G = 16
M = 16
K = 4
GM = G * M
BT = 512
NEG = -1e30


def _router_kernel(h_ref, wg_ref, wm_ref, fp_ref, gi_ref, aux_ref,
                   gp_acc, mp_acc):
    pid = pl.program_id(0)
    nb = pl.num_programs(0)

    h = h_ref[...]
    gl = jnp.dot(h, wg_ref[...], preferred_element_type=jnp.float32)
    ml = jnp.dot(h, wm_ref[...], preferred_element_type=jnp.float32)

    gmax = jnp.max(gl, axis=-1, keepdims=True)
    ge = jnp.exp(gl - gmax)
    gs = jnp.sum(ge, axis=-1, keepdims=True)
    gp = ge / gs
    top_gp = jnp.max(gp, axis=-1, keepdims=True)
    giota = jax.lax.broadcasted_iota(jnp.int32, gl.shape, 1)
    gidx = jnp.min(jnp.where(gp == top_gp, giota, G), axis=-1,
                   keepdims=True)

    ciota = jax.lax.broadcasted_iota(jnp.int32, ml.shape, 1)
    sel = (ciota // M) == gidx
    mlm = jnp.where(sel, ml, NEG)
    mmax = jnp.max(mlm, axis=-1, keepdims=True)
    me = jnp.exp(mlm - mmax)
    ms = jnp.sum(me, axis=-1, keepdims=True)
    mp = me / ms

    work = mp
    vals = []
    idxs = []
    for _ in range(K):
        v = jnp.max(work, axis=-1, keepdims=True)
        i = jnp.min(jnp.where(work == v, ciota, GM), axis=-1,
                    keepdims=True)
        vals.append(v)
        idxs.append(i)
        work = jnp.where(ciota == i, NEG, work)

    topv = jnp.concatenate(vals, axis=1)
    fin = top_gp * topv
    fin = fin / jnp.sum(fin, axis=-1, keepdims=True)
    fp_ref[...] = fin
    gi_ref[...] = jnp.concatenate(idxs, axis=1)

    gp_blk = jnp.sum(gp, axis=0, keepdims=True)
    mp_blk = jnp.sum(mp, axis=0, keepdims=True)

    @pl.when(pid == 0)
    def _init():
        gp_acc[...] = gp_blk
        mp_acc[...] = mp_blk

    @pl.when(pid != 0)
    def _accum():
        gp_acc[...] += gp_blk
        mp_acc[...] += mp_blk

    @pl.when(pid == nb - 1)
    def _finalize():
        gmean = gp_acc[...] / T
        gloss = jnp.sum(gmean * gmean)
        fr = jax.lax.broadcasted_iota(jnp.int32, (GM, M), 0) % M
        fc = jax.lax.broadcasted_iota(jnp.int32, (GM, M), 1)
        fold = (fr == fc).astype(jnp.float32)
        msum = jnp.dot(mp_acc[...], fold,
                       preferred_element_type=jnp.float32)
        mmean = msum / T
        mloss = jnp.sum(mmean * mmean)
        aux_ref[...] = jnp.reshape(gloss + mloss, (1, 1))


def kernel(hidden_states, group_gate_w, mini_gates):
    wg = group_gate_w.T
    wm = jnp.transpose(mini_gates, (1, 0, 2)).reshape(D, GM)

    nb = T // BT
    fp, gi, aux = pl.pallas_call(
        _router_kernel,
        grid=(nb,),
        in_specs=[
            pl.BlockSpec((BT, D), lambda i: (i, 0)),
            pl.BlockSpec((D, G), lambda i: (0, 0)),
            pl.BlockSpec((D, GM), lambda i: (0, 0)),
        ],
        out_specs=[
            pl.BlockSpec((BT, K), lambda i: (i, 0)),
            pl.BlockSpec((BT, K), lambda i: (i, 0)),
            pl.BlockSpec((1, 1), lambda i: (0, 0)),
        ],
        out_shape=[
            jax.ShapeDtypeStruct((T, K), jnp.float32),
            jax.ShapeDtypeStruct((T, K), jnp.int32),
            jax.ShapeDtypeStruct((1, 1), jnp.float32),
        ],
        scratch_shapes=[
            pltpu.VMEM((1, G), jnp.float32),
            pltpu.VMEM((1, GM), jnp.float32),
        ],
    )(hidden_states, wg, wm)

    return fp, gi, aux.reshape(())


# speedup vs baseline: 6.4369x; 1.1068x over previous
import jax
import jax.numpy as jnp
from jax.experimental import pallas as pl
from jax.experimental.pallas import tpu as pltpu

T = 8192
D = 1024
G = 16
M = 16
K = 4
GM = G * M
BT = 512
NEG = -1e30


def _router_kernel(h_ref, wg_ref, wm_ref, fp_ref, gi_ref, aux_ref,
                   gp_acc, mp_acc):
    pid = pl.program_id(0)
    nb = pl.num_programs(0)

    h = h_ref[...]
    gl = jnp.dot(h, wg_ref[...], preferred_element_type=jnp.float32)
    ml = jnp.dot(h, wm_ref[...], preferred_element_type=jnp.float32)

    gmax = jnp.max(gl, axis=-1, keepdims=True)
    ge = jnp.exp(gl - gmax)
    gs = jnp.sum(ge, axis=-1, keepdims=True)
    gp = ge / gs
    top_gp = jnp.max(gp, axis=-1, keepdims=True)
    giota = jax.lax.broadcasted_iota(jnp.int32, gl.shape, 1)
    gidx = jnp.min(jnp.where(gp == top_gp, giota, G), axis=-1,
                   keepdims=True)

    ciota = jax.lax.broadcasted_iota(jnp.int32, ml.shape, 1)
    sel = (ciota >> 4) == gidx
    mlm = jnp.where(sel, ml, NEG)
    mmax = jnp.max(mlm, axis=-1, keepdims=True)
    me = jnp.exp(mlm - mmax)
    ms = jnp.sum(me, axis=-1, keepdims=True)

    bits = jax.lax.bitcast_convert_type(me, jnp.int32)
    packed = (bits & jnp.int32(-256)) | (jnp.int32(GM - 1) - ciota)
    vals = []
    idxs = []
    for _ in range(K):
        pmax = jnp.max(packed, axis=-1, keepdims=True)
        idxs.append(jnp.int32(GM - 1) - (pmax & jnp.int32(GM - 1)))
        vals.append(jax.lax.bitcast_convert_type(
            pmax & jnp.int32(-256), jnp.float32))
        packed = jnp.where(packed == pmax, jnp.int32(-2147483647), packed)

    topv = jnp.concatenate(vals, axis=1)
    fp_ref[...] = topv / jnp.sum(topv, axis=-1, keepdims=True)
    gi_ref[...] = jnp.concatenate(idxs, axis=1)

    gp_blk = jnp.sum(gp, axis=0, keepdims=True)
    mp_blk = jnp.sum(me * (1.0 / ms), axis=0, keepdims=True)

    @pl.when(pid == 0)
    def _init():
        gp_acc[...] = gp_blk
        mp_acc[...] = mp_blk

    @pl.when(pid != 0)
    def _accum():
        gp_acc[...] += gp_blk
        mp_acc[...] += mp_blk

    @pl.when(pid == nb - 1)
    def _finalize():
        gmean = gp_acc[...] / T
        gloss = jnp.sum(gmean * gmean)
        fr = jax.lax.broadcasted_iota(jnp.int32, (GM, M), 0) % M
        fc = jax.lax.broadcasted_iota(jnp.int32, (GM, M), 1)
        fold = (fr == fc).astype(jnp.float32)
        msum = jnp.dot(mp_acc[...], fold,
                       preferred_element_type=jnp.float32)
        mmean = msum / T
        mloss = jnp.sum(mmean * mmean)
        aux_ref[...] = jnp.reshape(gloss + mloss, (1, 1))


def kernel(hidden_states, group_gate_w, mini_gates):
    wg = group_gate_w.T
    wm = jnp.transpose(mini_gates, (1, 0, 2)).reshape(D, GM)

    nb = T // BT
    fp, gi, aux = pl.pallas_call(
        _router_kernel,
        grid=(nb,),
        in_specs=[
            pl.BlockSpec((BT, D), lambda i: (i, 0)),
            pl.BlockSpec((D, G), lambda i: (0, 0)),
            pl.BlockSpec((D, GM), lambda i: (0, 0)),
        ],
        out_specs=[
            pl.BlockSpec((BT, K), lambda i: (i, 0)),
            pl.BlockSpec((BT, K), lambda i: (i, 0)),
            pl.BlockSpec((1, 1), lambda i: (0, 0)),
        ],
        out_shape=[
            jax.ShapeDtypeStruct((T, K), jnp.float32),
            jax.ShapeDtypeStruct((T, K), jnp.int32),
            jax.ShapeDtypeStruct((1, 1), jnp.float32),
        ],
        scratch_shapes=[
            pltpu.VMEM((1, G), jnp.float32),
            pltpu.VMEM((1, GM), jnp.float32),
        ],
    )(hidden_states, wg, wm)

    return fp, gi, aux.reshape(())


# speedup vs baseline: 6.6509x; 1.0333x over previous
import jax
import jax.numpy as jnp
from jax.experimental import pallas as pl
from jax.experimental.pallas import tpu as pltpu

T = 8192
D = 1024
G = 16
M = 16
K = 4
GM = G * M
BT = 512
NEG = -1e30


def _router_kernel(h_ref, w_ref, fp_ref, gi_ref, aux_ref,
                   gp_acc, mp_acc):
    pid = pl.program_id(0)
    nb = pl.num_programs(0)

    h = h_ref[...]
    out = jnp.dot(h, w_ref[...], preferred_element_type=jnp.float32)
    ml = out[:, :GM]
    gl = out[:, GM:]

    gmax = jnp.max(gl, axis=-1, keepdims=True)
    ge = jnp.exp(gl - gmax)
    gs = jnp.sum(ge, axis=-1, keepdims=True)
    gp = ge / gs
    top_gp = jnp.max(gp, axis=-1, keepdims=True)
    giota = jax.lax.broadcasted_iota(jnp.int32, gl.shape, 1)
    gidx = jnp.min(jnp.where(gp == top_gp, giota, G), axis=-1,
                   keepdims=True)

    ciota = jax.lax.broadcasted_iota(jnp.int32, ml.shape, 1)
    sel = (ciota >> 4) == gidx
    mlm = jnp.where(sel, ml, NEG)
    mmax = jnp.max(mlm, axis=-1, keepdims=True)
    me = jnp.exp(mlm - mmax)
    ms = jnp.sum(me, axis=-1, keepdims=True)

    bits = jax.lax.bitcast_convert_type(me, jnp.int32)
    packed = (bits & jnp.int32(-256)) | (jnp.int32(GM - 1) - ciota)
    packed = jnp.maximum(packed[:, :GM // 2], packed[:, GM // 2:])
    vals = []
    idxs = []
    for k in range(K):
        pmax = jnp.max(packed, axis=-1, keepdims=True)
        idxs.append(jnp.int32(GM - 1) - (pmax & jnp.int32(GM - 1)))
        vals.append(jax.lax.bitcast_convert_type(
            pmax & jnp.int32(-256), jnp.float32))
        if k < K - 1:
            packed = jnp.where(packed == pmax,
                               jnp.int32(-2147483647), packed)

    topv = jnp.concatenate(vals, axis=1)
    fp_ref[...] = topv / jnp.sum(topv, axis=-1, keepdims=True)
    gi_ref[...] = jnp.concatenate(idxs, axis=1)

    gp_blk = jnp.sum(gp, axis=0, keepdims=True)
    mp_blk = jnp.sum(me * (1.0 / ms), axis=0, keepdims=True)

    @pl.when(pid == 0)
    def _init():
        gp_acc[...] = gp_blk
        mp_acc[...] = mp_blk

    @pl.when(pid != 0)
    def _accum():
        gp_acc[...] += gp_blk
        mp_acc[...] += mp_blk

    @pl.when(pid == nb - 1)
    def _finalize():
        gmean = gp_acc[...] / T
        gloss = jnp.sum(gmean * gmean)
        fr = jax.lax.broadcasted_iota(jnp.int32, (GM, M), 0) % M
        fc = jax.lax.broadcasted_iota(jnp.int32, (GM, M), 1)
        fold = (fr == fc).astype(jnp.float32)
        msum = jnp.dot(mp_acc[...], fold,
                       preferred_element_type=jnp.float32)
        mmean = msum / T
        mloss = jnp.sum(mmean * mmean)
        aux_ref[...] = jnp.reshape(gloss + mloss, (1, 1))


def kernel(hidden_states, group_gate_w, mini_gates):
    wm = jnp.transpose(mini_gates, (1, 0, 2)).reshape(D, GM)
    w = jnp.concatenate([wm, group_gate_w.T], axis=1)

    nb = T // BT
    fp, gi, aux = pl.pallas_call(
        _router_kernel,
        grid=(nb,),
        in_specs=[
            pl.BlockSpec((BT, D), lambda i: (i, 0)),
            pl.BlockSpec((D, GM + G), lambda i: (0, 0)),
        ],
        out_specs=[
            pl.BlockSpec((BT, K), lambda i: (i, 0)),
            pl.BlockSpec((BT, K), lambda i: (i, 0)),
            pl.BlockSpec((1, 1), lambda i: (0, 0)),
        ],
        out_shape=[
            jax.ShapeDtypeStruct((T, K), jnp.float32),
            jax.ShapeDtypeStruct((T, K), jnp.int32),
            jax.ShapeDtypeStruct((1, 1), jnp.float32),
        ],
        scratch_shapes=[
            pltpu.VMEM((1, G), jnp.float32),
            pltpu.VMEM((1, GM), jnp.float32),
        ],
    )(hidden_states, w)

    return fp, gi, aux.reshape(())


# speedup vs baseline: 7.5131x; 1.1296x over previous
import jax
import jax.numpy as jnp
from jax.experimental import pallas as pl
from jax.experimental.pallas import tpu as pltpu

T = 8192
D = 1024
G = 16
M = 16
K = 4
GM = G * M
BT = 1024
NEG = -1e30


def _router_kernel(h_ref, w_ref, fp_ref, gi_ref, aux_ref,
                   gp_acc, mp_acc):
    pid = pl.program_id(0)
    nb = pl.num_programs(0)

    h = h_ref[...]
    out = jnp.dot(h, w_ref[...], preferred_element_type=jnp.float32)
    ml = out[:, :GM]
    gl = out[:, GM:]

    gmax = jnp.max(gl, axis=-1, keepdims=True)
    ge = jnp.exp(gl - gmax)
    gs = jnp.sum(ge, axis=-1, keepdims=True)
    gp = ge / gs
    top_gp = jnp.max(gp, axis=-1, keepdims=True)
    giota = jax.lax.broadcasted_iota(jnp.int32, gl.shape, 1)
    gidx = jnp.min(jnp.where(gp == top_gp, giota, G), axis=-1,
                   keepdims=True)

    ciota = jax.lax.broadcasted_iota(jnp.int32, ml.shape, 1)
    sel = (ciota >> 4) == gidx
    mlm = jnp.where(sel, ml, NEG)
    mmax = jnp.max(mlm, axis=-1, keepdims=True)
    me = jnp.exp(mlm - mmax)
    ms = jnp.sum(me, axis=-1, keepdims=True)

    bits = jax.lax.bitcast_convert_type(me, jnp.int32)
    packed = (bits & jnp.int32(-256)) | (jnp.int32(GM - 1) - ciota)
    packed = jnp.maximum(packed[:, :GM // 2], packed[:, GM // 2:])
    vals = []
    idxs = []
    for k in range(K):
        pmax = jnp.max(packed, axis=-1, keepdims=True)
        idxs.append(jnp.int32(GM - 1) - (pmax & jnp.int32(GM - 1)))
        vals.append(jax.lax.bitcast_convert_type(
            pmax & jnp.int32(-256), jnp.float32))
        if k < K - 1:
            packed = jnp.where(packed == pmax,
                               jnp.int32(-2147483647), packed)

    topv = jnp.concatenate(vals, axis=1)
    fp_ref[...] = topv / jnp.sum(topv, axis=-1, keepdims=True)
    gi_ref[...] = jnp.concatenate(idxs, axis=1)

    gp_blk = jnp.sum(gp, axis=0, keepdims=True)
    mp_blk = jnp.sum(me * (1.0 / ms), axis=0, keepdims=True)

    @pl.when(pid == 0)
    def _init():
        gp_acc[...] = gp_blk
        mp_acc[...] = mp_blk

    @pl.when(pid != 0)
    def _accum():
        gp_acc[...] += gp_blk
        mp_acc[...] += mp_blk

    @pl.when(pid == nb - 1)
    def _finalize():
        gmean = gp_acc[...] / T
        gloss = jnp.sum(gmean * gmean)
        fr = jax.lax.broadcasted_iota(jnp.int32, (GM, M), 0) % M
        fc = jax.lax.broadcasted_iota(jnp.int32, (GM, M), 1)
        fold = (fr == fc).astype(jnp.float32)
        msum = jnp.dot(mp_acc[...], fold,
                       preferred_element_type=jnp.float32)
        mmean = msum / T
        mloss = jnp.sum(mmean * mmean)
        aux_ref[...] = jnp.reshape(gloss + mloss, (1, 1))


def kernel(hidden_states, group_gate_w, mini_gates):
    wm = jnp.transpose(mini_gates, (1, 0, 2)).reshape(D, GM)
    w = jnp.concatenate([wm, group_gate_w.T], axis=1)

    nb = T // BT
    fp, gi, aux = pl.pallas_call(
        _router_kernel,
        grid=(nb,),
        in_specs=[
            pl.BlockSpec((BT, D), lambda i: (i, 0)),
            pl.BlockSpec((D, GM + G), lambda i: (0, 0)),
        ],
        out_specs=[
            pl.BlockSpec((BT, K), lambda i: (i, 0)),
            pl.BlockSpec((BT, K), lambda i: (i, 0)),
            pl.BlockSpec((1, 1), lambda i: (0, 0)),
        ],
        out_shape=[
            jax.ShapeDtypeStruct((T, K), jnp.float32),
            jax.ShapeDtypeStruct((T, K), jnp.int32),
            jax.ShapeDtypeStruct((1, 1), jnp.float32),
        ],
        scratch_shapes=[
            pltpu.VMEM((1, G), jnp.float32),
            pltpu.VMEM((1, GM), jnp.float32),
        ],
    )(hidden_states, w)

    return fp, gi, aux.reshape(())
